# baseline (speedup 1.0000x reference)
import functools

import jax
import jax.numpy as jnp
from jax import lax
from jax.experimental import pallas as pl
from jax.experimental.pallas import tpu as pltpu
from jax.experimental.pallas import tpu_sc as plsc

NUM_GROUP_K = 512
GROUP_SIZE_K = 32
ROW_PAD = 16
SEG = 64
NSEG = NUM_GROUP_K // SEG
NCHUNK = 32
CAP = 768
RCH = 8
SELBLK = 256


def _fps_seg_body(seg, x_ref, y_ref, z_ref, d_ref, s_ref,
                  cx_ref, cy_ref, cz_ref, do_ref, so_ref):
    B, N = x_ref.shape
    x = x_ref[...]
    y = y_ref[...]
    z = z_ref[...]
    flane = lax.broadcasted_iota(jnp.int32, (B, N), 1).astype(jnp.float32)
    gcol = lax.broadcasted_iota(jnp.int32, (B, SEG), 1)
    bigf = jnp.float32(2.0 * N)
    lx0 = s_ref[:, 0:1]
    ly0 = s_ref[:, 1:2]
    lz0 = s_ref[:, 2:3]
    if seg == 0:
        do_ref[...] = jnp.full((B, N), jnp.inf, dtype=jnp.float32)
        cx0 = jnp.where(gcol == 0, lx0, 0.0)
        cy0 = jnp.where(gcol == 0, ly0, 0.0)
        cz0 = jnp.where(gcol == 0, lz0, 0.0)
        j0 = 1
    else:
        do_ref[...] = d_ref[...]
        cx0 = jnp.zeros((B, SEG), jnp.float32)
        cy0 = jnp.zeros((B, SEG), jnp.float32)
        cz0 = jnp.zeros((B, SEG), jnp.float32)
        j0 = 0

    def step(j, carry):
        lx, ly, lz, cx, cy, cz = carry
        dx = x - lx
        dy = y - ly
        dz = z - lz
        d = (dx * dx + dy * dy) + dz * dz
        dist = jnp.minimum(do_ref[...], d)
        do_ref[...] = dist
        mx = jnp.max(dist, axis=1, keepdims=True)
        nxt = jnp.min(jnp.where(dist == mx, flane, bigf), axis=1, keepdims=True)
        sel = flane == nxt
        lx = jnp.sum(jnp.where(sel, x, 0.0), axis=1, keepdims=True)
        ly = jnp.sum(jnp.where(sel, y, 0.0), axis=1, keepdims=True)
        lz = jnp.sum(jnp.where(sel, z, 0.0), axis=1, keepdims=True)
        hit = gcol == j
        cx = cx + jnp.where(hit, lx, 0.0)
        cy = cy + jnp.where(hit, ly, 0.0)
        cz = cz + jnp.where(hit, lz, 0.0)
        return lx, ly, lz, cx, cy, cz

    lx, ly, lz, cx, cy, cz = lax.fori_loop(
        j0, SEG, step, (lx0, ly0, lz0, cx0, cy0, cz0))
    cx_ref[...] = cx
    cy_ref[...] = cy
    cz_ref[...] = cz
    so_ref[...] = jnp.concatenate(
        [lx, ly, lz, jnp.zeros((B, 5), jnp.float32)], axis=1)


def _fps_segment(seg, x, y, z, dist, state):
    B, N = x.shape
    outc = jax.ShapeDtypeStruct((B, SEG), jnp.float32)
    return pl.pallas_call(
        functools.partial(_fps_seg_body, seg),
        out_shape=(outc, outc, outc,
                   jax.ShapeDtypeStruct((B, N), jnp.float32),
                   jax.ShapeDtypeStruct((B, 8), jnp.float32)),
        input_output_aliases={3: 3, 4: 4},
    )(x, y, z, dist, state)


def _dist_body(x_ref, y_ref, z_ref, c_ref, d_ref, t_ref):
    N = x_ref.shape[2]
    x = x_ref[0]
    y = y_ref[0]
    z = z_ref[0]
    c = c_ref[0]
    dx = c[:, 0:1] - x
    dy = c[:, 1:2] - y
    dz = c[:, 2:3] - z
    d0 = (dx * dx + dy * dy) + dz * dz
    d_ref[0] = d0
    cm = d0[:, 0:128]
    for ci in range(1, N // 128):
        cm = jnp.minimum(cm, d0[:, ci * 128:(ci + 1) * 128])
    cm = jnp.minimum(cm[:, 0:64], cm[:, 64:128])
    cm = jnp.minimum(cm[:, 0:NCHUNK], cm[:, NCHUNK:64])
    t_ref[0] = jnp.max(cm, axis=1, keepdims=True)


def _knn_dist(x3, y3, z3, center):
    B = x3.shape[0]
    N = x3.shape[2]
    return pl.pallas_call(
        _dist_body,
        grid=(B, 1),
        in_specs=[
            pl.BlockSpec((1, 1, N), lambda b, g: (b, 0, 0)),
            pl.BlockSpec((1, 1, N), lambda b, g: (b, 0, 0)),
            pl.BlockSpec((1, 1, N), lambda b, g: (b, 0, 0)),
            pl.BlockSpec((1, SEG, 3), lambda b, g: (b, 0, 0)),
        ],
        out_specs=(
            pl.BlockSpec((1, SEG, N), lambda b, g: (b, 0, 0)),
            pl.BlockSpec((1, SEG, 1), lambda b, g: (b, 0, 0)),
        ),
        out_shape=(
            jax.ShapeDtypeStruct((B, SEG, N), jnp.float32),
            jax.ShapeDtypeStruct((B, SEG, 1), jnp.float32),
        ),
    )(x3, y3, z3, center)


def _sc_compress(dflat, tflat, R2, N):
    info = plsc.get_sparse_core_info()
    nw = info.num_cores * info.num_subcores
    rpw = R2 // nw
    mesh = plsc.VectorSubcoreMesh(core_axis_name="c", subcore_axis_name="s")
    capb = CAP + 16
    rch = min(RCH, rpw)

    @functools.partial(
        pl.kernel,
        mesh=mesh,
        compiler_params=pltpu.CompilerParams(use_tc_tiling_on_sc=False,
                                             needs_layout_passes=False),
        out_type=(
            jax.ShapeDtypeStruct((R2, CAP), jnp.float32),
            jax.ShapeDtypeStruct((R2, CAP), jnp.float32),
        ),
        scratch_types=[
            pltpu.VMEM((rch * N,), jnp.float32),
            pltpu.VMEM((capb,), jnp.float32),
            pltpu.VMEM((capb,), jnp.float32),
            pltpu.VMEM((rpw * 16,), jnp.float32),
            pltpu.VMEM((16,), jnp.int32),
        ],
    )
    def k(d_hbm, t_hbm, cd_hbm, ci_hbm, rows_v, cd_v, ci_v, t_v, ptr_v):
        wid = lax.axis_index("s") * info.num_cores + lax.axis_index("c")
        rbase = wid * rpw
        pltpu.sync_copy(t_hbm.at[pl.ds(rbase * 16, rpw * 16)], t_v)
        iota16 = lax.iota(jnp.int32, 16)
        one16 = jnp.full((16,), 1, jnp.int32)
        zero16 = jnp.zeros((16,), jnp.int32)
        inf16 = jnp.full((16,), jnp.inf, jnp.float32)
        big16 = jnp.full((16,), jnp.float32(3 * N), jnp.float32)
        maxpos = jnp.full((16,), capb - 16, jnp.int32)

        def chunk_body(cix, _):
            pltpu.sync_copy(
                d_hbm.at[pl.ds((rbase + cix * rch) * N, rch * N)], rows_v)

            def row_body(rl, _):
                def pf(i, _):
                    cd_v[pl.ds(i * 16, 16)] = inf16
                    ci_v[pl.ds(i * 16, 16)] = big16
                    return 0
                lax.fori_loop(0, capb // 16, pf, 0)
                tvec = t_v[pl.ds((cix * rch + rl) * 16, 16)]
                ptr_v[...] = jnp.zeros((16,), jnp.int32)

                def scan_body(i, _):
                    for u in range(4):
                        iu = i * 4 + u
                        v = rows_v[pl.ds(rl * N + iu * 16, 16)]
                        mask = v <= tvec

                        @pl.when(jnp.any(mask))
                        def _():
                            ptr = ptr_v[...]
                            cs = plsc.cumsum(jnp.where(mask, one16, zero16))
                            pos = jnp.minimum(ptr + cs - 1, maxpos)
                            posf = (iota16 + iu * 16).astype(jnp.float32)
                            plsc.store_scatter(cd_v, [pos], v, mask=mask)
                            plsc.store_scatter(ci_v, [pos], posf, mask=mask)
                            ptr_v[...] = ptr + \
                                plsc.all_reduce_population_count(mask)
                    return 0

                lax.fori_loop(0, N // 64, scan_body, 0)
                r = rbase + cix * rch + rl
                pltpu.sync_copy(cd_v.at[pl.ds(0, CAP)], cd_hbm.at[r])
                pltpu.sync_copy(ci_v.at[pl.ds(0, CAP)], ci_hbm.at[r])
                return 0

            lax.fori_loop(0, rch, row_body, 0)
            return 0

        lax.fori_loop(0, rpw // rch, chunk_body, 0)

    return k(dflat, tflat)


def _sel_body(cd_ref, ci_ref, idx_ref, ds_ref):
    M = idx_ref.shape[2]
    d0 = cd_ref[0]
    idxf = ci_ref[0]
    ds_ref[...] = d0
    mcol = lax.broadcasted_iota(jnp.int32, (SELBLK, M), 1)
    bigf = jnp.float32(3.0 * 8192)
    mn0 = jnp.min(d0, axis=1, keepdims=True)

    def body(j, carry):
        acc, mn = carry
        dcur = ds_ref[...]
        am = jnp.min(jnp.where(dcur == mn, idxf, bigf), axis=1, keepdims=True)
        dnew = jnp.where(idxf == am, jnp.inf, dcur)
        ds_ref[...] = dnew
        mn2 = jnp.min(dnew, axis=1, keepdims=True)
        return acc + jnp.where(mcol == j, am.astype(jnp.int32), 0), mn2

    acc, _ = lax.fori_loop(
        0, M, body, (jnp.zeros((SELBLK, M), jnp.int32), mn0))
    idx_ref[0] = acc


def _topk_sel(cd3, ci3):
    R = cd3.shape[0]
    M = GROUP_SIZE_K
    cd4 = cd3.reshape(R // SELBLK, SELBLK, CAP)
    ci4 = ci3.reshape(R // SELBLK, SELBLK, CAP)
    return pl.pallas_call(
        _sel_body,
        grid=(R // SELBLK,),
        in_specs=[
            pl.BlockSpec((1, SELBLK, CAP), lambda b: (b, 0, 0)),
            pl.BlockSpec((1, SELBLK, CAP), lambda b: (b, 0, 0)),
        ],
        out_specs=pl.BlockSpec((1, SELBLK, M), lambda b: (b, 0, 0)),
        out_shape=jax.ShapeDtypeStruct((R // SELBLK, SELBLK, M), jnp.int32),
        scratch_shapes=[pltpu.VMEM((SELBLK, CAP), jnp.float32)],
    )(cd4, ci4)


def _sc_gather_normalize(flat_idx, pts_pad, cent_pad):
    R = flat_idx.shape[0]
    info = plsc.get_sparse_core_info()
    nw = info.num_cores * info.num_subcores
    rpw = R // nw
    gpw = rpw // GROUP_SIZE_K
    mesh = plsc.VectorSubcoreMesh(core_axis_name="c", subcore_axis_name="s")

    @functools.partial(
        pl.kernel,
        mesh=mesh,
        compiler_params=pltpu.CompilerParams(use_tc_tiling_on_sc=False),
        out_type=jax.ShapeDtypeStruct((R, ROW_PAD), jnp.float32),
        scratch_types=[
            pltpu.VMEM((rpw,), jnp.int32),
            pltpu.VMEM((rpw, ROW_PAD), jnp.float32),
            pltpu.VMEM((gpw, ROW_PAD), jnp.float32),
            pltpu.SemaphoreType.DMA,
        ],
    )
    def k(idx_hbm, pts_hbm, cent_hbm, out_hbm, idx_v, rows_v, cent_v, sem):
        wid = lax.axis_index("s") * info.num_cores + lax.axis_index("c")
        rbase = wid * rpw
        pltpu.sync_copy(idx_hbm.at[pl.ds(rbase, rpw)], idx_v)
        pltpu.async_copy(pts_hbm.at[idx_v], rows_v, sem).wait()
        pltpu.sync_copy(cent_hbm.at[pl.ds(wid * gpw, gpw)], cent_v)

        def body(g, _):
            cvec = cent_v[g]
            base = g * GROUP_SIZE_K
            for j in range(GROUP_SIZE_K):
                rows_v[base + j] = rows_v[base + j] - cvec
            return 0

        lax.fori_loop(0, gpw, body, 0)
        pltpu.sync_copy(rows_v, out_hbm.at[pl.ds(rbase, rpw)])

    return k(flat_idx, pts_pad, cent_pad)


def kernel(pts):
    B, N, C = pts.shape
    G = NUM_GROUP_K
    M = GROUP_SIZE_K
    x = pts[:, :, 0]
    y = pts[:, :, 1]
    z = pts[:, :, 2]
    x3 = x[:, None, :]
    y3 = y[:, None, :]
    z3 = z[:, None, :]
    dist = jnp.zeros((B, N), jnp.float32)
    state = jnp.pad(pts[:, 0, :3], ((0, 0), (0, 5)))
    cxs, cys, czs, cds, cis = [], [], [], [], []
    for seg in range(NSEG):
        cx, cy, cz, dist, state = _fps_segment(
            0 if seg == 0 else 1, x, y, z, dist, state)
        cxs.append(cx)
        cys.append(cy)
        czs.append(cz)
        cseg = jnp.stack([cx, cy, cz], axis=-1)
        dseg, tseg = _knn_dist(x3, y3, z3, cseg)
        t16 = jnp.broadcast_to(tseg.reshape(B * SEG, 1), (B * SEG, 16))
        cdseg, ciseg = _sc_compress(
            dseg.reshape(B * SEG * N), t16.reshape(B * SEG * 16),
            B * SEG, N)
        cds.append(cdseg)
        cis.append(ciseg)
    center = jnp.concatenate(
        [jnp.stack([cxs[s], cys[s], czs[s]], axis=-1) for s in range(NSEG)],
        axis=1)
    cd_all = jnp.stack(cds, 0).reshape(NSEG, B, SEG, CAP).transpose(
        1, 0, 2, 3).reshape(B * G, CAP)
    ci_all = jnp.stack(cis, 0).reshape(NSEG, B, SEG, CAP).transpose(
        1, 0, 2, 3).reshape(B * G, CAP)
    idx = _topk_sel(cd_all, ci_all).reshape(B, G, M)
    flat_idx = (idx + jnp.arange(B, dtype=jnp.int32)[:, None, None] * N
                ).reshape(B * G * M)
    pts_pad = jnp.pad(pts.reshape(B * N, C), ((0, 0), (0, ROW_PAD - C)))
    cent_pad = jnp.pad(center.reshape(B * G, 3), ((0, 0), (0, ROW_PAD - 3)))
    rows = _sc_gather_normalize(flat_idx, pts_pad, cent_pad)
    neighborhood = rows[:, :C].reshape(B, G, M, C)
    return neighborhood, center


# speedup vs baseline: 1.7331x; 1.7331x over previous
import functools

import jax
import jax.numpy as jnp
from jax import lax
from jax.experimental import pallas as pl
from jax.experimental.pallas import tpu as pltpu
from jax.experimental.pallas import tpu_sc as plsc

NUM_GROUP_K = 512
GROUP_SIZE_K = 32
ROW_PAD = 16
GBLK = 256


def _fps_body(x_ref, y_ref, z_ref, cx_ref, cy_ref, cz_ref, dist_ref):
    B, N = x_ref.shape
    G = cx_ref.shape[1]
    x = x_ref[...]
    y = y_ref[...]
    z = z_ref[...]
    flane = lax.broadcasted_iota(jnp.int32, (B, N), 1).astype(jnp.float32)
    gcol = lax.broadcasted_iota(jnp.int32, (B, G), 1)
    bigf = jnp.float32(2.0 * N)

    dist_ref[...] = jnp.full((B, N), jnp.inf, dtype=jnp.float32)
    lx0 = x[:, 0:1]
    ly0 = y[:, 0:1]
    lz0 = z[:, 0:1]
    cx0 = jnp.where(gcol == 0, lx0, 0.0)
    cy0 = jnp.where(gcol == 0, ly0, 0.0)
    cz0 = jnp.where(gcol == 0, lz0, 0.0)

    def step(j, carry):
        lx, ly, lz, cx, cy, cz = carry
        dx = x - lx
        dy = y - ly
        dz = z - lz
        d = (dx * dx + dy * dy) + dz * dz
        dist = jnp.minimum(dist_ref[...], d)
        dist_ref[...] = dist
        mx = jnp.max(dist, axis=1, keepdims=True)
        nxt = jnp.min(jnp.where(dist == mx, flane, bigf), axis=1, keepdims=True)
        sel = flane == nxt
        lx = jnp.sum(jnp.where(sel, x, 0.0), axis=1, keepdims=True)
        ly = jnp.sum(jnp.where(sel, y, 0.0), axis=1, keepdims=True)
        lz = jnp.sum(jnp.where(sel, z, 0.0), axis=1, keepdims=True)
        hit = gcol == j
        cx = cx + jnp.where(hit, lx, 0.0)
        cy = cy + jnp.where(hit, ly, 0.0)
        cz = cz + jnp.where(hit, lz, 0.0)
        return lx, ly, lz, cx, cy, cz

    _, _, _, cx, cy, cz = lax.fori_loop(
        1, G, step, (lx0, ly0, lz0, cx0, cy0, cz0))
    cx_ref[...] = cx
    cy_ref[...] = cy
    cz_ref[...] = cz


def _fps_centers(x, y, z):
    B, N = x.shape
    G = NUM_GROUP_K
    out = jax.ShapeDtypeStruct((B, G), jnp.float32)
    return pl.pallas_call(
        _fps_body,
        out_shape=(out, out, out),
        scratch_shapes=[pltpu.VMEM((B, N), jnp.float32)],
    )(x, y, z)


def _knn_body(x_ref, y_ref, z_ref, c_ref, idx_ref, d_ref):
    N = x_ref.shape[2]
    M = idx_ref.shape[2]
    x = x_ref[0]
    y = y_ref[0]
    z = z_ref[0]
    c = c_ref[0]
    dx = c[:, 0:1] - x
    dy = c[:, 1:2] - y
    dz = c[:, 2:3] - z
    d0 = (dx * dx + dy * dy) + dz * dz
    d_ref[...] = d0
    flane = lax.broadcasted_iota(jnp.int32, (GBLK, N), 1).astype(jnp.float32)
    mcol = lax.broadcasted_iota(jnp.int32, (GBLK, M), 1)
    bigf = jnp.float32(2.0 * N)
    mn0 = jnp.min(d0, axis=1, keepdims=True)

    def body(j, carry):
        acc, mn = carry
        dcur = d_ref[...]
        am = jnp.min(jnp.where(dcur == mn, flane, bigf), axis=1, keepdims=True)
        dnew = jnp.where(flane == am, jnp.inf, dcur)
        d_ref[...] = dnew
        mn2 = jnp.min(dnew, axis=1, keepdims=True)
        return acc + jnp.where(mcol == j, am.astype(jnp.int32), 0), mn2

    acc, _ = lax.fori_loop(
        0, M, body, (jnp.zeros((GBLK, M), jnp.int32), mn0))
    idx_ref[0] = acc


def _knn_topk(x, y, z, center):
    B, N = x.shape
    G = NUM_GROUP_K
    M = GROUP_SIZE_K
    grid = (B, G // GBLK)
    x3 = x[:, None, :]
    y3 = y[:, None, :]
    z3 = z[:, None, :]
    return pl.pallas_call(
        _knn_body,
        grid=grid,
        in_specs=[
            pl.BlockSpec((1, 1, N), lambda b, g: (b, 0, 0)),
            pl.BlockSpec((1, 1, N), lambda b, g: (b, 0, 0)),
            pl.BlockSpec((1, 1, N), lambda b, g: (b, 0, 0)),
            pl.BlockSpec((1, GBLK, 3), lambda b, g: (b, g, 0)),
        ],
        out_specs=pl.BlockSpec((1, GBLK, M), lambda b, g: (b, g, 0)),
        out_shape=jax.ShapeDtypeStruct((B, G, M), jnp.int32),
        scratch_shapes=[pltpu.VMEM((GBLK, N), jnp.float32)],
    )(x3, y3, z3, center)


def _sc_gather_normalize(flat_idx, pts_pad, cent_pad):
    R = flat_idx.shape[0]
    info = plsc.get_sparse_core_info()
    nw = info.num_cores * info.num_subcores
    rpw = R // nw
    gpw = rpw // GROUP_SIZE_K
    mesh = plsc.VectorSubcoreMesh(core_axis_name="c", subcore_axis_name="s")

    @functools.partial(
        pl.kernel,
        mesh=mesh,
        compiler_params=pltpu.CompilerParams(use_tc_tiling_on_sc=False),
        out_type=jax.ShapeDtypeStruct((R, ROW_PAD), jnp.float32),
        scratch_types=[
            pltpu.VMEM((rpw,), jnp.int32),
            pltpu.VMEM((rpw, ROW_PAD), jnp.float32),
            pltpu.VMEM((gpw, ROW_PAD), jnp.float32),
            pltpu.SemaphoreType.DMA,
        ],
    )
    def k(idx_hbm, pts_hbm, cent_hbm, out_hbm, idx_v, rows_v, cent_v, sem):
        wid = lax.axis_index("s") * info.num_cores + lax.axis_index("c")
        rbase = wid * rpw
        pltpu.sync_copy(idx_hbm.at[pl.ds(rbase, rpw)], idx_v)
        pltpu.async_copy(pts_hbm.at[idx_v], rows_v, sem).wait()
        pltpu.sync_copy(cent_hbm.at[pl.ds(wid * gpw, gpw)], cent_v)

        def body(g, _):
            cvec = cent_v[g]
            base = g * GROUP_SIZE_K
            for j in range(GROUP_SIZE_K):
                rows_v[base + j] = rows_v[base + j] - cvec
            return 0

        lax.fori_loop(0, gpw, body, 0)
        pltpu.sync_copy(rows_v, out_hbm.at[pl.ds(rbase, rpw)])

    return k(flat_idx, pts_pad, cent_pad)


def kernel(pts):
    B, N, C = pts.shape
    G = NUM_GROUP_K
    M = GROUP_SIZE_K
    x = pts[:, :, 0]
    y = pts[:, :, 1]
    z = pts[:, :, 2]
    cx, cy, cz = _fps_centers(x, y, z)
    center = jnp.stack([cx, cy, cz], axis=-1)
    idx = _knn_topk(x, y, z, center)
    flat_idx = (idx + jnp.arange(B, dtype=jnp.int32)[:, None, None] * N
                ).reshape(B * G * M)
    pts_pad = jnp.pad(pts.reshape(B * N, C), ((0, 0), (0, ROW_PAD - C)))
    cent_pad = jnp.pad(center.reshape(B * G, 3), ((0, 0), (0, ROW_PAD - 3)))
    rows = _sc_gather_normalize(flat_idx, pts_pad, cent_pad)
    neighborhood = rows[:, :C].reshape(B, G, M, C)
    return neighborhood, center
